# Initial kernel scaffold; baseline (speedup 1.0000x reference)
import jax
import jax.numpy as jnp
from jax.experimental import pallas as pl


def kernel(nf, edge_index, ef, W1, b1, Wr1, br1, Wr2, br2):
    raise NotImplementedError("write your pallas kernel here")


# speedup vs baseline: 4.3779x; 4.3779x over previous
import functools

import jax
import jax.numpy as jnp
from jax import lax
from jax.experimental import pallas as pl
from jax.experimental.pallas import tpu as pltpu
from jax.experimental.pallas import tpu_sc as plsc

N = 10000
E = 320000
ND = 128
ED = 4
HD = 128
TD = 1
ALPHA = 0.1
TOL = 1e-05
MAX_ITER = 20

NC = 2
NS = 16
NW = NC * NS
CHUNK = 128
NCHUNK_W = 80
E_PAD = NW * NCHUNK_W * CHUNK
AGG_ROWS = 10240
ZROWS = AGG_ROWS // NS

BM = 1000
GRID = N // BM


def _make_seg_kernel(width):
    mesh = plsc.VectorSubcoreMesh(core_axis_name="c", subcore_axis_name="s")

    @functools.partial(
        pl.kernel,
        out_type=jax.ShapeDtypeStruct((NC, AGG_ROWS, width), jnp.float32),
        mesh=mesh,
        scratch_types=[
            pltpu.VMEM((NCHUNK_W, CHUNK), jnp.int32),
            pltpu.VMEM((NCHUNK_W, CHUNK), jnp.int32),
            pltpu.VMEM((CHUNK, width), jnp.float32),
            pltpu.VMEM_SHARED((AGG_ROWS, width), jnp.float32),
            pltpu.SemaphoreType.DMA,
        ],
    )
    def seg(table_hbm, src_hbm, dst_hbm, zeros_hbm, out_hbm,
            srcv, dstv, rows, aggsh, gsem):
        cid = lax.axis_index("c")
        sid = lax.axis_index("s")
        wid = sid * NC + cid

        pltpu.sync_copy(zeros_hbm, aggsh.at[pl.ds(sid * ZROWS, ZROWS)])
        pltpu.sync_copy(src_hbm.at[pl.ds(wid * NCHUNK_W, NCHUNK_W)], srcv)
        pltpu.sync_copy(dst_hbm.at[pl.ds(wid * NCHUNK_W, NCHUNK_W)], dstv)
        plsc.subcore_barrier()

        def body(j, carry):
            pltpu.async_copy(table_hbm.at[srcv.at[j]], rows, gsem).wait()
            pltpu.sync_copy(rows, aggsh.at[dstv.at[j]], add=True)
            return carry

        lax.fori_loop(0, NCHUNK_W, body, 0)
        plsc.subcore_barrier()
        pltpu.sync_copy(aggsh.at[pl.ds(sid * ZROWS, ZROWS)],
                        out_hbm.at[cid, pl.ds(sid * ZROWS, ZROWS)])

    return seg


_seg128 = _make_seg_kernel(HD)


BME = 8192


def _efw_body(ef_ref, w1e_ref, out_ref):
    out_ref[...] = jnp.dot(ef_ref[...], w1e_ref[...],
                           preferred_element_type=jnp.float32)


def _compute_efw(ef_pad, w1e):
    return pl.pallas_call(
        _efw_body,
        grid=(E_PAD // BME,),
        in_specs=[
            pl.BlockSpec((BME, ED), lambda i: (i, 0)),
            pl.BlockSpec((ED, HD), lambda i: (0, 0)),
        ],
        out_specs=pl.BlockSpec((BME, HD), lambda i: (i, 0)),
        out_shape=jax.ShapeDtypeStruct((E_PAD, HD), jnp.float32),
    )(ef_pad, w1e)


def _c_body(nf, anf0, anf1, aef0, aef1, w1a, w1n, b1, c_ref):
    acc = jnp.dot(nf[...], w1a[...], preferred_element_type=jnp.float32)
    acc += jnp.dot(anf0[0] + anf1[0], w1n[...], preferred_element_type=jnp.float32)
    c_ref[...] = acc + aef0[0] + aef1[0] + b1[...]


def _compute_c(nf, aggnf, aggefw, w1a, w1n, b1):
    return pl.pallas_call(
        _c_body,
        grid=(GRID,),
        in_specs=[
            pl.BlockSpec((BM, ND), lambda i: (i, 0)),
            pl.BlockSpec((1, BM, ND), lambda i: (0, i, 0)),
            pl.BlockSpec((1, BM, ND), lambda i: (1, i, 0)),
            pl.BlockSpec((1, BM, HD), lambda i: (0, i, 0)),
            pl.BlockSpec((1, BM, HD), lambda i: (1, i, 0)),
            pl.BlockSpec((ND, HD), lambda i: (0, 0)),
            pl.BlockSpec((ND, HD), lambda i: (0, 0)),
            pl.BlockSpec((1, HD), lambda i: (0, 0)),
        ],
        out_specs=pl.BlockSpec((BM, HD), lambda i: (i, 0)),
        out_shape=jax.ShapeDtypeStruct((N, HD), jnp.float32),
    )(nf, aggnf, aggnf, aggefw, aggefw, w1a, w1n, b1)


def _iter_body(a0, a1, c, h, w1h, hn_ref, n2_ref):
    agg = a0[0] + a1[0]
    hn = jnp.maximum(
        jnp.dot(agg, w1h[...], preferred_element_type=jnp.float32) + c[...], 0.0)
    hnew = (1.0 - ALPHA) * h[...] + ALPHA * hn
    hn_ref[...] = hnew
    d = h[...] - hnew
    s = jnp.sum(d * d, axis=0)
    row = lax.broadcasted_iota(jnp.int32, (1, 8, HD), 1)
    n2_ref[...] = jnp.where(row == 0, s[None, None, :], 0.0)


def _iter_tc(agg, c, h, w1h):
    return pl.pallas_call(
        _iter_body,
        grid=(GRID,),
        in_specs=[
            pl.BlockSpec((1, BM, HD), lambda i: (0, i, 0)),
            pl.BlockSpec((1, BM, HD), lambda i: (1, i, 0)),
            pl.BlockSpec((BM, HD), lambda i: (i, 0)),
            pl.BlockSpec((BM, HD), lambda i: (i, 0)),
            pl.BlockSpec((HD, HD), lambda i: (0, 0)),
        ],
        out_specs=[
            pl.BlockSpec((BM, HD), lambda i: (i, 0)),
            pl.BlockSpec((1, 8, HD), lambda i: (i, 0, 0)),
        ],
        out_shape=[
            jax.ShapeDtypeStruct((N, HD), jnp.float32),
            jax.ShapeDtypeStruct((GRID, 8, HD), jnp.float32),
        ],
    )(agg, agg, c, h, w1h)


def _final_body(a0, a1, c, w1h, wr1, br1, wr2, br2, out_ref):
    agg = a0[0] + a1[0]
    hf = jnp.maximum(
        jnp.dot(agg, w1h[...], preferred_element_type=jnp.float32) + c[...], 0.0)
    hr = jnp.maximum(
        jnp.dot(hf, wr1[...], preferred_element_type=jnp.float32) + br1[...], 0.0)
    out_ref[...] = jnp.dot(hr, wr2[...], preferred_element_type=jnp.float32) + br2[...]


def _final_tc(agg, c, w1h, wr1, br1, wr2, br2):
    return pl.pallas_call(
        _final_body,
        grid=(GRID,),
        in_specs=[
            pl.BlockSpec((1, BM, HD), lambda i: (0, i, 0)),
            pl.BlockSpec((1, BM, HD), lambda i: (1, i, 0)),
            pl.BlockSpec((BM, HD), lambda i: (i, 0)),
            pl.BlockSpec((HD, HD), lambda i: (0, 0)),
            pl.BlockSpec((HD, HD), lambda i: (0, 0)),
            pl.BlockSpec((1, HD), lambda i: (0, 0)),
            pl.BlockSpec((HD, TD), lambda i: (0, 0)),
            pl.BlockSpec((1, TD), lambda i: (0, 0)),
        ],
        out_specs=pl.BlockSpec((BM, TD), lambda i: (i, 0)),
        out_shape=jax.ShapeDtypeStruct((N, TD), jnp.float32),
    )(agg, agg, c, w1h, wr1, br1, wr2, br2)


def kernel(nf, edge_index, ef, W1, b1, Wr1, br1, Wr2, br2):
    src = edge_index[0]
    dst = edge_index[1]
    pad = E_PAD - E
    src_p = jnp.concatenate([src, jnp.zeros((pad,), jnp.int32)]).reshape(-1, CHUNK)
    dst_p = jnp.concatenate([dst, jnp.full((pad,), N, jnp.int32)]).reshape(-1, CHUNK)

    zeros128 = jnp.zeros((ZROWS, HD), jnp.float32)

    w1a = W1[:ND]
    w1h = W1[ND:ND + HD]
    w1n = W1[ND + HD:ND + HD + ND]
    w1e = W1[ND + HD + ND:]

    aggnf = _seg128(nf, src_p, dst_p, zeros128)
    ef_pad = jnp.zeros((E_PAD, ED), jnp.float32).at[:E].set(ef)
    efw = _compute_efw(ef_pad, w1e)
    iota_p = jnp.arange(E_PAD, dtype=jnp.int32).reshape(-1, CHUNK)
    aggefw = _seg128(efw, iota_p, dst_p, zeros128)
    c = _compute_c(nf, aggnf, aggefw, w1a, w1n, b1.reshape(1, HD))

    tol2 = jnp.float32(TOL) * jnp.float32(TOL)

    def cond(state):
        i, _, done = state
        return jnp.logical_and(i < MAX_ITER, jnp.logical_not(done))

    def body(state):
        i, h, _ = state
        agg = _seg128(h, src_p, dst_p, zeros128)
        hnew, n2 = _iter_tc(agg, c, h, w1h)
        done = jnp.sum(n2) < tol2
        h = jnp.where(done, h, hnew)
        return (i + 1, h, done)

    h0 = jnp.zeros((N, HD), jnp.float32)
    _, h, _ = lax.while_loop(cond, body, (jnp.int32(0), h0, jnp.array(False)))

    agg = _seg128(h, src_p, dst_p, zeros128)
    return _final_tc(agg, c, w1h, Wr1, br1.reshape(1, HD), Wr2, br2.reshape(1, TD))


# speedup vs baseline: 4.9079x; 1.1211x over previous
import functools

import jax
import jax.numpy as jnp
from jax import lax
from jax.experimental import pallas as pl
from jax.experimental.pallas import tpu as pltpu
from jax.experimental.pallas import tpu_sc as plsc

N = 10000
E = 320000
ND = 128
ED = 4
HD = 128
TD = 1
ALPHA = 0.1
TOL = 1e-05
MAX_ITER = 20

NC = 2
NS = 16
NW = NC * NS
CHUNK = 128
NCHUNK_W = 80
E_PAD = NW * NCHUNK_W * CHUNK
AGG_ROWS = 10240
ZROWS = AGG_ROWS // NS

BM = 1000
GRID = N // BM


def _make_seg_kernel(width):
    mesh = plsc.VectorSubcoreMesh(core_axis_name="c", subcore_axis_name="s")

    @functools.partial(
        pl.kernel,
        out_type=jax.ShapeDtypeStruct((NC, AGG_ROWS, width), jnp.float32),
        mesh=mesh,
        scratch_types=[
            pltpu.VMEM((NCHUNK_W // 2, CHUNK), jnp.int32),
            pltpu.VMEM((NCHUNK_W // 2, CHUNK), jnp.int32),
            pltpu.VMEM((2, CHUNK, width), jnp.float32),
            pltpu.VMEM_SHARED((AGG_ROWS, width), jnp.float32),
            pltpu.SemaphoreType.DMA,
            pltpu.SemaphoreType.DMA,
        ],
    )
    def seg(table_hbm, src_hbm, dst_hbm, zeros_hbm, out_hbm,
            srcv, dstv, rows, aggsh, gsem0, gsem1):
        cid = lax.axis_index("c")
        sid = lax.axis_index("s")
        wid = sid * NC + cid

        pltpu.sync_copy(zeros_hbm, aggsh.at[pl.ds(sid * ZROWS, ZROWS)])
        plsc.subcore_barrier()

        cpp = NCHUNK_W // 2
        for p in range(2):
            pltpu.sync_copy(
                src_hbm.at[pl.ds(wid * NCHUNK_W + p * cpp, cpp)], srcv)
            pltpu.sync_copy(
                dst_hbm.at[pl.ds(wid * NCHUNK_W + p * cpp, cpp)], dstv)

            pltpu.async_copy(table_hbm.at[srcv.at[0]], rows.at[0], gsem0)

            def body(k, carry):
                j0 = 2 * k
                j1 = j0 + 1
                pltpu.async_copy(table_hbm.at[srcv.at[j1]], rows.at[1], gsem1)
                pltpu.make_async_copy(
                    table_hbm.at[srcv.at[j0]], rows.at[0], gsem0).wait()
                pltpu.sync_copy(rows.at[0], aggsh.at[dstv.at[j0]], add=True)

                @pl.when(j0 + 2 < cpp)
                def _():
                    pltpu.async_copy(
                        table_hbm.at[srcv.at[j0 + 2]], rows.at[0], gsem0)

                pltpu.make_async_copy(
                    table_hbm.at[srcv.at[j1]], rows.at[1], gsem1).wait()
                pltpu.sync_copy(rows.at[1], aggsh.at[dstv.at[j1]], add=True)
                return carry

            lax.fori_loop(0, cpp // 2, body, 0)

        plsc.subcore_barrier()
        pltpu.sync_copy(aggsh.at[pl.ds(sid * ZROWS, ZROWS)],
                        out_hbm.at[cid, pl.ds(sid * ZROWS, ZROWS)])

    return seg


_seg128 = _make_seg_kernel(HD)


BME = 8192


def _efw_body(ef_ref, w1e_ref, out_ref):
    out_ref[...] = jnp.dot(ef_ref[...], w1e_ref[...],
                           preferred_element_type=jnp.float32, precision=lax.Precision.HIGHEST)


def _compute_efw(ef_pad, w1e):
    return pl.pallas_call(
        _efw_body,
        grid=(E_PAD // BME,),
        in_specs=[
            pl.BlockSpec((BME, ED), lambda i: (i, 0)),
            pl.BlockSpec((ED, HD), lambda i: (0, 0)),
        ],
        out_specs=pl.BlockSpec((BME, HD), lambda i: (i, 0)),
        out_shape=jax.ShapeDtypeStruct((E_PAD, HD), jnp.float32),
    )(ef_pad, w1e)


def _c_body(nf, anf0, anf1, aef0, aef1, w1a, w1n, b1, c_ref):
    acc = jnp.dot(nf[...], w1a[...], preferred_element_type=jnp.float32, precision=lax.Precision.HIGHEST)
    acc += jnp.dot(anf0[0] + anf1[0], w1n[...], preferred_element_type=jnp.float32, precision=lax.Precision.HIGHEST)
    c_ref[...] = acc + aef0[0] + aef1[0] + b1[...]


def _compute_c(nf, aggnf, aggefw, w1a, w1n, b1):
    return pl.pallas_call(
        _c_body,
        grid=(GRID,),
        in_specs=[
            pl.BlockSpec((BM, ND), lambda i: (i, 0)),
            pl.BlockSpec((1, BM, ND), lambda i: (0, i, 0)),
            pl.BlockSpec((1, BM, ND), lambda i: (1, i, 0)),
            pl.BlockSpec((1, BM, HD), lambda i: (0, i, 0)),
            pl.BlockSpec((1, BM, HD), lambda i: (1, i, 0)),
            pl.BlockSpec((ND, HD), lambda i: (0, 0)),
            pl.BlockSpec((ND, HD), lambda i: (0, 0)),
            pl.BlockSpec((1, HD), lambda i: (0, 0)),
        ],
        out_specs=pl.BlockSpec((BM, HD), lambda i: (i, 0)),
        out_shape=jax.ShapeDtypeStruct((N, HD), jnp.float32),
    )(nf, aggnf, aggnf, aggefw, aggefw, w1a, w1n, b1)


def _iter_body(a0, a1, c, h, w1h, hn_ref, n2_ref):
    agg = a0[0] + a1[0]
    hn = jnp.maximum(
        jnp.dot(agg, w1h[...], preferred_element_type=jnp.float32, precision=lax.Precision.HIGHEST) + c[...], 0.0)
    hnew = (1.0 - ALPHA) * h[...] + ALPHA * hn
    hn_ref[...] = hnew
    d = h[...] - hnew
    s = jnp.sum(d * d, axis=0)
    row = lax.broadcasted_iota(jnp.int32, (1, 8, HD), 1)
    n2_ref[...] = jnp.where(row == 0, s[None, None, :], 0.0)


def _iter_tc(agg, c, h, w1h):
    return pl.pallas_call(
        _iter_body,
        grid=(GRID,),
        in_specs=[
            pl.BlockSpec((1, BM, HD), lambda i: (0, i, 0)),
            pl.BlockSpec((1, BM, HD), lambda i: (1, i, 0)),
            pl.BlockSpec((BM, HD), lambda i: (i, 0)),
            pl.BlockSpec((BM, HD), lambda i: (i, 0)),
            pl.BlockSpec((HD, HD), lambda i: (0, 0)),
        ],
        out_specs=[
            pl.BlockSpec((BM, HD), lambda i: (i, 0)),
            pl.BlockSpec((1, 8, HD), lambda i: (i, 0, 0)),
        ],
        out_shape=[
            jax.ShapeDtypeStruct((N, HD), jnp.float32),
            jax.ShapeDtypeStruct((GRID, 8, HD), jnp.float32),
        ],
    )(agg, agg, c, h, w1h)


def _final_body(a0, a1, c, w1h, wr1, br1, wr2, br2, out_ref):
    agg = a0[0] + a1[0]
    hf = jnp.maximum(
        jnp.dot(agg, w1h[...], preferred_element_type=jnp.float32, precision=lax.Precision.HIGHEST) + c[...], 0.0)
    hr = jnp.maximum(
        jnp.dot(hf, wr1[...], preferred_element_type=jnp.float32, precision=lax.Precision.HIGHEST) + br1[...], 0.0)
    out_ref[...] = jnp.dot(hr, wr2[...], preferred_element_type=jnp.float32, precision=lax.Precision.HIGHEST) + br2[...]


def _final_tc(agg, c, w1h, wr1, br1, wr2, br2):
    return pl.pallas_call(
        _final_body,
        grid=(GRID,),
        in_specs=[
            pl.BlockSpec((1, BM, HD), lambda i: (0, i, 0)),
            pl.BlockSpec((1, BM, HD), lambda i: (1, i, 0)),
            pl.BlockSpec((BM, HD), lambda i: (i, 0)),
            pl.BlockSpec((HD, HD), lambda i: (0, 0)),
            pl.BlockSpec((HD, HD), lambda i: (0, 0)),
            pl.BlockSpec((1, HD), lambda i: (0, 0)),
            pl.BlockSpec((HD, TD), lambda i: (0, 0)),
            pl.BlockSpec((1, TD), lambda i: (0, 0)),
        ],
        out_specs=pl.BlockSpec((BM, TD), lambda i: (i, 0)),
        out_shape=jax.ShapeDtypeStruct((N, TD), jnp.float32),
    )(agg, agg, c, w1h, wr1, br1, wr2, br2)


def kernel(nf, edge_index, ef, W1, b1, Wr1, br1, Wr2, br2):
    src = edge_index[0]
    dst = edge_index[1]
    pad = E_PAD - E
    src_p = jnp.concatenate([src, jnp.zeros((pad,), jnp.int32)]).reshape(-1, CHUNK)
    dst_p = jnp.concatenate([dst, jnp.full((pad,), N, jnp.int32)]).reshape(-1, CHUNK)

    zeros128 = jnp.zeros((ZROWS, HD), jnp.float32)

    w1a = W1[:ND]
    w1h = W1[ND:ND + HD]
    w1n = W1[ND + HD:ND + HD + ND]
    w1e = W1[ND + HD + ND:]

    aggnf = _seg128(nf, src_p, dst_p, zeros128)
    ef_pad = jnp.zeros((E_PAD, ED), jnp.float32).at[:E].set(ef)
    efw = _compute_efw(ef_pad, w1e)
    iota_p = jnp.arange(E_PAD, dtype=jnp.int32).reshape(-1, CHUNK)
    aggefw = _seg128(efw, iota_p, dst_p, zeros128)
    c = _compute_c(nf, aggnf, aggefw, w1a, w1n, b1.reshape(1, HD))

    tol2 = jnp.float32(TOL) * jnp.float32(TOL)

    def cond(state):
        i, _, done = state
        return jnp.logical_and(i < MAX_ITER, jnp.logical_not(done))

    def body(state):
        i, h, _ = state
        agg = _seg128(h, src_p, dst_p, zeros128)
        hnew, n2 = _iter_tc(agg, c, h, w1h)
        done = jnp.sum(n2) < tol2
        h = jnp.where(done, h, hnew)
        return (i + 1, h, done)

    h0 = jnp.zeros((N, HD), jnp.float32)
    _, h, _ = lax.while_loop(cond, body, (jnp.int32(0), h0, jnp.array(False)))

    agg = _seg128(h, src_p, dst_p, zeros128)
    return _final_tc(agg, c, w1h, Wr1, br1.reshape(1, HD), Wr2, br2.reshape(1, TD))
